# baseline (speedup 1.0000x reference)
import functools

import jax
import jax.numpy as jnp
from jax import lax
from jax.experimental import pallas as pl
from jax.experimental.pallas import tpu as pltpu
from jax.experimental.pallas import tpu_sc as plsc

_NC = 2
_NS = 16
_NW = _NC * _NS
_LANES = 16
_NIN = 8
_NMF = 3
_HTBL = 96
_TILE = 1408


def _pack4(i0, i1, i2, i3):
    return ((i0 * _NMF + i1) * _NMF + i2) * _NMF + i3


def _sc_body(nrp, rows, xf_hbm, idx_hbm, out_hbm,
             xrows, atbl, btbl, idxtile, hitile, lotile, bufs, outsems,
             idxsem):
    cid = lax.axis_index("c")
    sid = lax.axis_index("s")
    wid = sid * _NC + cid
    base = wid * rows

    pltpu.sync_copy(xf_hbm.at[pl.ds(base * (_NIN * _NMF), rows * _NIN * _NMF)],
                    xrows)

    def build_tables(c, g):
        bl = g // _HTBL
        a = g % _HTBL
        xoff = bl * (_NIN * _NMF)
        d0 = a // 27
        d1 = (a // 9) % _NMF
        d2 = (a // _NMF) % _NMF
        d3 = a % _NMF
        for tbl, ibase in ((atbl, 0), (btbl, 4)):
            cb = xoff + (ibase * _NMF)
            v0 = plsc.load_gather(xrows, [cb + d0])
            v1 = plsc.load_gather(xrows, [cb + _NMF + d1])
            v2 = plsc.load_gather(xrows, [cb + 2 * _NMF + d2])
            v3 = plsc.load_gather(xrows, [cb + 3 * _NMF + d3])
            tbl[pl.ds(c * _LANES, _LANES)] = (v0 * v1) * (v2 * v3)
        return g + _LANES

    lax.fori_loop(0, rows * _HTBL // _LANES, build_tables,
                  lax.iota(jnp.int32, _LANES))

    ntiles = nrp // _TILE
    pending = [None, None]
    idx_pending = pltpu.async_copy(idx_hbm.at[:, pl.ds(0, _TILE)],
                                   idxtile, idxsem)
    for t in range(ntiles):
        r0 = t * _TILE
        buf = bufs[t % 2]
        sem = outsems[t % 2]
        if pending[t % 2] is not None:
            pending[t % 2].wait()
            pending[t % 2] = None
        idx_pending.wait()

        @plsc.parallel_loop(0, _TILE, step=_LANES, unroll=4)
        def pack_codes(off):
            iv = [idxtile[i, pl.ds(off, _LANES)] for i in range(_NIN)]
            hitile[pl.ds(off, _LANES)] = _pack4(iv[0], iv[1], iv[2], iv[3])
            lotile[pl.ds(off, _LANES)] = _pack4(iv[4], iv[5], iv[6], iv[7])

        if t + 1 < ntiles:
            idx_pending = pltpu.async_copy(
                idx_hbm.at[:, pl.ds(r0 + _TILE, _TILE)], idxtile, idxsem)

        @plsc.parallel_loop(0, _TILE, step=_LANES, unroll=4)
        def gather_mul(off):
            hi = hitile[pl.ds(off, _LANES)]
            lo = lotile[pl.ds(off, _LANES)]
            for bl in range(rows):
                av = plsc.load_gather(atbl, [hi + (bl * _HTBL)])
                bv = plsc.load_gather(btbl, [lo + (bl * _HTBL)])
                buf[bl, pl.ds(off, _LANES)] = av * bv

        pending[t % 2] = pltpu.async_copy(
            buf, out_hbm.at[pl.ds(base, rows), pl.ds(r0, _TILE)], sem)

    for p in pending:
        if p is not None:
            p.wait()


def kernel(x, mf_indices):
    B, n_in, n_mfs = x.shape
    n_rules = mf_indices.shape[0]
    rows = B // _NW
    nrp = pl.cdiv(n_rules, _TILE) * _TILE
    xf = x.reshape(B * n_in * n_mfs)
    idxT = mf_indices.astype(jnp.int32).T
    idxT = jnp.pad(idxT, ((0, 0), (0, nrp - n_rules)))

    mesh = plsc.VectorSubcoreMesh(core_axis_name="c", subcore_axis_name="s")
    body = functools.partial(_sc_body, nrp, rows)
    f = pl.kernel(
        body,
        out_type=jax.ShapeDtypeStruct((B, nrp), jnp.float32),
        mesh=mesh,
        scratch_types=dict(
            xrows=pltpu.VMEM((rows * n_in * n_mfs,), jnp.float32),
            atbl=pltpu.VMEM((rows * _HTBL,), jnp.float32),
            btbl=pltpu.VMEM((rows * _HTBL,), jnp.float32),
            idxtile=pltpu.VMEM((n_in, _TILE), jnp.int32),
            hitile=pltpu.VMEM((_TILE,), jnp.int32),
            lotile=pltpu.VMEM((_TILE,), jnp.int32),
            bufs=[pltpu.VMEM((rows, _TILE), jnp.float32) for _ in range(2)],
            outsems=[pltpu.SemaphoreType.DMA for _ in range(2)],
            idxsem=pltpu.SemaphoreType.DMA,
        ),
        compiler_params=pltpu.CompilerParams(
            needs_layout_passes=False, disable_bounds_checks=True),
    )
    out = f(xf, idxT)
    return out[:, :n_rules]


# speedup vs baseline: 1.2076x; 1.2076x over previous
import functools

import jax
import jax.numpy as jnp
from jax import lax
from jax.experimental import pallas as pl
from jax.experimental.pallas import tpu as pltpu
from jax.experimental.pallas import tpu_sc as plsc

_NC = 2
_NS = 16
_NW = _NC * _NS
_LANES = 16
_NIN = 8
_NMF = 3
_HTBL = 96
_TILE = 1664


def _pack4(i0, i1, i2, i3):
    return ((i0 * _NMF + i1) * _NMF + i2) * _NMF + i3


def _sc_body(nrp, rows, xf_hbm, idx_hbm, out_hbm,
             xrows, atbl, btbl, idxtile, hitile, lotile, bufs, outsems,
             idxsem):
    cid = lax.axis_index("c")
    sid = lax.axis_index("s")
    wid = sid * _NC + cid
    base = wid * rows

    pltpu.sync_copy(xf_hbm.at[pl.ds(base * (_NIN * _NMF), rows * _NIN * _NMF)],
                    xrows)

    def build_tables(c, g):
        bl = g // _HTBL
        a = g % _HTBL
        xoff = bl * (_NIN * _NMF)
        d0 = a // 27
        d1 = (a // 9) % _NMF
        d2 = (a // _NMF) % _NMF
        d3 = a % _NMF
        for tbl, ibase in ((atbl, 0), (btbl, 4)):
            cb = xoff + (ibase * _NMF)
            v0 = plsc.load_gather(xrows, [cb + d0])
            v1 = plsc.load_gather(xrows, [cb + _NMF + d1])
            v2 = plsc.load_gather(xrows, [cb + 2 * _NMF + d2])
            v3 = plsc.load_gather(xrows, [cb + 3 * _NMF + d3])
            tbl[pl.ds(c * _LANES, _LANES)] = (v0 * v1) * (v2 * v3)
        return g + _LANES

    lax.fori_loop(0, rows * _HTBL // _LANES, build_tables,
                  lax.iota(jnp.int32, _LANES))

    ntiles = nrp // _TILE
    pending = [None, None]
    idx_pending = pltpu.async_copy(idx_hbm.at[:, pl.ds(0, _TILE)],
                                   idxtile, idxsem)
    for t in range(ntiles):
        r0 = t * _TILE
        buf = bufs[t % 2]
        sem = outsems[t % 2]
        if pending[t % 2] is not None:
            pending[t % 2].wait()
            pending[t % 2] = None
        idx_pending.wait()

        @plsc.parallel_loop(0, _TILE, step=_LANES, unroll=2)
        def pack_codes(off):
            iv = [idxtile[i, pl.ds(off, _LANES)] for i in range(_NIN)]
            hitile[pl.ds(off, _LANES)] = _pack4(iv[0], iv[1], iv[2], iv[3])
            lotile[pl.ds(off, _LANES)] = _pack4(iv[4], iv[5], iv[6], iv[7])

        if t + 1 < ntiles:
            idx_pending = pltpu.async_copy(
                idx_hbm.at[:, pl.ds(r0 + _TILE, _TILE)], idxtile, idxsem)

        @plsc.parallel_loop(0, _TILE, step=_LANES, unroll=2)
        def gather_mul(off):
            hi = hitile[pl.ds(off, _LANES)]
            lo = lotile[pl.ds(off, _LANES)]
            for bl in range(rows):
                av = plsc.load_gather(atbl, [hi + (bl * _HTBL)])
                bv = plsc.load_gather(btbl, [lo + (bl * _HTBL)])
                buf[bl, pl.ds(off, _LANES)] = av * bv

        pending[t % 2] = pltpu.async_copy(
            buf, out_hbm.at[pl.ds(base, rows), pl.ds(r0, _TILE)], sem)

    for p in pending:
        if p is not None:
            p.wait()


def kernel(x, mf_indices):
    B, n_in, n_mfs = x.shape
    n_rules = mf_indices.shape[0]
    rows = B // _NW
    nrp = pl.cdiv(n_rules, _TILE) * _TILE
    xf = x.reshape(B * n_in * n_mfs)
    idxT = mf_indices.astype(jnp.int32).T
    idxT = jnp.pad(idxT, ((0, 0), (0, nrp - n_rules)))

    mesh = plsc.VectorSubcoreMesh(core_axis_name="c", subcore_axis_name="s")
    body = functools.partial(_sc_body, nrp, rows)
    f = pl.kernel(
        body,
        out_type=jax.ShapeDtypeStruct((B, nrp), jnp.float32),
        mesh=mesh,
        scratch_types=dict(
            xrows=pltpu.VMEM((rows * n_in * n_mfs,), jnp.float32),
            atbl=pltpu.VMEM((rows * _HTBL,), jnp.float32),
            btbl=pltpu.VMEM((rows * _HTBL,), jnp.float32),
            idxtile=pltpu.VMEM((n_in, _TILE), jnp.int32),
            hitile=pltpu.VMEM((_TILE,), jnp.int32),
            lotile=pltpu.VMEM((_TILE,), jnp.int32),
            bufs=[pltpu.VMEM((rows, _TILE), jnp.float32) for _ in range(2)],
            outsems=[pltpu.SemaphoreType.DMA for _ in range(2)],
            idxsem=pltpu.SemaphoreType.DMA,
        ),
        compiler_params=pltpu.CompilerParams(
            needs_layout_passes=False, disable_bounds_checks=True),
    )
    out = f(xf, idxT)
    return out[:, :n_rules]
